# baseline (speedup 1.0000x reference)
import functools

import jax
import jax.numpy as jnp
from jax import lax
from jax.experimental import pallas as pl
from jax.experimental.pallas import tpu as pltpu
from jax.experimental.pallas import tpu_sc as plsc

N = 10000
E = 320000
F_IN = 128
H = 16
C = 10
G = 128

NB = 1024
GRID_N = 10

NC = 2
NS = 16
NW = NC * NS
FG = 4
EG = NW // FG
FPW = H // FG
EPG = E // EG
ECH = 8000
DCH = 2000
EPW = E // NW

_mesh = lambda: plsc.VectorSubcoreMesh(
    core_axis_name="c", subcore_axis_name="s", num_cores=NC, num_subcores=NS)


def _sc_deg_body(ei_hbm, out_hbm, dst_buf, acc):
    wid = lax.axis_index("s") * NC + lax.axis_index("c")
    zero = jnp.zeros((16,), jnp.float32)
    ones = jnp.ones((16,), jnp.float32)

    @plsc.parallel_loop(0, N // 16, unroll=8)
    def _(i):
        acc[pl.ds(i * 16, 16)] = zero

    base = wid * EPW
    for ch in range(EPW // DCH):
        pltpu.sync_copy(ei_hbm.at[pl.ds(E + base + ch * DCH, DCH)], dst_buf)

        @plsc.parallel_loop(0, DCH // 16, unroll=8)
        def _(i):
            d16 = dst_buf[pl.ds(i * 16, 16)]
            plsc.addupdate_scatter(acc, [d16], ones)

    pltpu.sync_copy(acc, out_hbm.at[wid])


@functools.cache
def _sc_deg():
    return pl.kernel(
        _sc_deg_body,
        out_type=jax.ShapeDtypeStruct((NW, N), jnp.float32),
        mesh=_mesh(),
        scratch_types=[
            pltpu.VMEM((DCH,), jnp.int32),
            pltpu.VMEM((N,), jnp.float32),
        ],
        compiler_params=pltpu.CompilerParams(needs_layout_passes=False),
    )


def _sc_edge_body(ei_hbm, hpT_hbm, out_hbm, src_buf0, src_buf1, dst_buf0, dst_buf1, table, acc, sems):
    wid = lax.axis_index("s") * NC + lax.axis_index("c")
    fg = wid // EG
    eg = wid % EG

    pltpu.sync_copy(hpT_hbm.at[pl.ds(fg * FPW, FPW), :], table)

    zero = jnp.zeros((16,), jnp.float32)

    @plsc.parallel_loop(0, N // 16, unroll=8)
    def _(i):
        for j in range(FPW):
            acc[j, pl.ds(i * 16, 16)] = zero

    jv = [jnp.full((16,), j, jnp.int32) for j in range(FPW)]
    base = eg * EPG
    nch = EPG // ECH
    sbufs = [src_buf0, src_buf1]
    dbufs = [dst_buf0, dst_buf1]
    cps = [pltpu.make_async_copy(ei_hbm.at[pl.ds(base + ch * ECH, ECH)],
                                 sbufs[ch % 2], sems.at[ch % 2, 0])
           for ch in range(nch)]
    cpd = [pltpu.make_async_copy(ei_hbm.at[pl.ds(E + base + ch * ECH, ECH)],
                                 dbufs[ch % 2], sems.at[ch % 2, 1])
           for ch in range(nch)]
    cps[0].start()
    cpd[0].start()
    for ch in range(nch):
        cps[ch].wait()
        cpd[ch].wait()
        if ch + 1 < nch:
            cps[ch + 1].start()
            cpd[ch + 1].start()
        sb = sbufs[ch % 2]
        db = dbufs[ch % 2]

        @plsc.parallel_loop(0, ECH // 16, unroll=8)
        def _(i):
            s16 = sb[pl.ds(i * 16, 16)]
            d16 = db[pl.ds(i * 16, 16)]
            for j in range(FPW):
                v = plsc.load_gather(table, [jv[j], s16])
                plsc.addupdate_scatter(acc, [jv[j], d16], v)

    pltpu.sync_copy(acc, out_hbm.at[eg, pl.ds(fg * FPW, FPW), :])


@functools.cache
def _sc_edge():
    return pl.kernel(
        _sc_edge_body,
        out_type=jax.ShapeDtypeStruct((EG, H, N), jnp.float32),
        mesh=_mesh(),
        scratch_types=[
            pltpu.VMEM((ECH,), jnp.int32),
            pltpu.VMEM((ECH,), jnp.int32),
            pltpu.VMEM((ECH,), jnp.int32),
            pltpu.VMEM((ECH,), jnp.int32),
            pltpu.VMEM((FPW, N), jnp.float32),
            pltpu.VMEM((FPW, N), jnp.float32),
            pltpu.SemaphoreType.DMA((2, 2)),
        ],
        compiler_params=pltpu.CompilerParams(needs_layout_passes=False),
    )


def _tc_prea_body(x_ref, w1_ref, hT_ref):
    h = jnp.dot(x_ref[...], w1_ref[...], preferred_element_type=jnp.float32)
    hT_ref[...] = h.T


def _tc_prea(x, W1):
    return pl.pallas_call(
        _tc_prea_body,
        grid=(GRID_N,),
        in_specs=[
            pl.BlockSpec((NB, F_IN), lambda i: (i, 0)),
            pl.BlockSpec((F_IN, H), lambda i: (0, 0)),
        ],
        out_specs=pl.BlockSpec((H, NB), lambda i: (0, i)),
        out_shape=jax.ShapeDtypeStruct((H, N), jnp.float32),
    )(x, W1)


def _tc_preb_body(degp_ref, hT_ref, hpT_ref, dinvT_ref):
    deg = jnp.sum(degp_ref[...], axis=0) + 1.0
    dinv = lax.rsqrt(deg)[None, :]
    hpT_ref[...] = hT_ref[...] * dinv
    dinvT_ref[...] = dinv


def _tc_preb(degp, h1T):
    return pl.pallas_call(
        _tc_preb_body,
        grid=(GRID_N,),
        in_specs=[
            pl.BlockSpec((NW, NB), lambda i: (0, i)),
            pl.BlockSpec((H, NB), lambda i: (0, i)),
        ],
        out_specs=[
            pl.BlockSpec((H, NB), lambda i: (0, i)),
            pl.BlockSpec((1, NB), lambda i: (0, i)),
        ],
        out_shape=[
            jax.ShapeDtypeStruct((H, N), jnp.float32),
            jax.ShapeDtypeStruct((1, N), jnp.float32),
        ],
    )(degp, h1T)


def _tc_mid_body(o1_ref, hpT_ref, dinvT_ref, b1_ref, w2_ref, hp2T_ref):
    s1 = jnp.sum(o1_ref[...], axis=0)
    dinv = dinvT_ref[...]
    h1 = jnp.maximum(dinv * (s1 + hpT_ref[...]) + b1_ref[...], 0.0)
    h2p = jnp.dot(w2_ref[...], h1, preferred_element_type=jnp.float32)
    hp2T_ref[...] = h2p * dinv


def _tc_mid(o1, hp1T, dinvT, b1c, W2):
    return pl.pallas_call(
        _tc_mid_body,
        grid=(GRID_N,),
        in_specs=[
            pl.BlockSpec((EG, H, NB), lambda i: (0, 0, i)),
            pl.BlockSpec((H, NB), lambda i: (0, i)),
            pl.BlockSpec((1, NB), lambda i: (0, i)),
            pl.BlockSpec((H, 1), lambda i: (0, 0)),
            pl.BlockSpec((H, H), lambda i: (0, 0)),
        ],
        out_specs=pl.BlockSpec((H, NB), lambda i: (0, i)),
        out_shape=jax.ShapeDtypeStruct((H, N), jnp.float32),
    )(o1, hp1T, dinvT, b1c, W2)


def _tc_post_body(o2_ref, hpT_ref, dinvT_ref, b2_ref, batch_ref, wfc_ref,
                  bfc_ref, out_ref, pooled):
    i = pl.program_id(0)
    s2 = jnp.sum(o2_ref[...], axis=0)
    dinv = dinvT_ref[...]
    h2 = dinv * (s2 + hpT_ref[...]) + b2_ref[...]

    b = batch_ref[...]
    gids = lax.broadcasted_iota(jnp.int32, (G, NB), 0)
    nidx = lax.broadcasted_iota(jnp.int32, (G, NB), 1) + i * NB
    mask = (b == gids) & (nidx < N)
    neg = jnp.float32(-jnp.inf)
    cols = [jnp.max(jnp.where(mask, h2[f:f + 1, :], neg), axis=1, keepdims=True)
            for f in range(H)]
    cont = jnp.concatenate(cols, axis=1)

    @pl.when(i == 0)
    def _():
        pooled[...] = jnp.full((G, H), neg, jnp.float32)

    pooled[...] = jnp.maximum(pooled[...], cont)

    @pl.when(i == pl.num_programs(0) - 1)
    def _():
        logits = jnp.dot(pooled[...], wfc_ref[...],
                         preferred_element_type=jnp.float32) + bfc_ref[...]
        m = jnp.max(logits, axis=1, keepdims=True)
        lse = jnp.log(jnp.sum(jnp.exp(logits - m), axis=1, keepdims=True))
        out_ref[...] = logits - m - lse


def _tc_post(o2, hp2T, dinvT, b2c, batch2d, Wfc, bfcc):
    return pl.pallas_call(
        _tc_post_body,
        grid=(GRID_N,),
        in_specs=[
            pl.BlockSpec((EG, H, NB), lambda i: (0, 0, i)),
            pl.BlockSpec((H, NB), lambda i: (0, i)),
            pl.BlockSpec((1, NB), lambda i: (0, i)),
            pl.BlockSpec((H, 1), lambda i: (0, 0)),
            pl.BlockSpec((1, NB), lambda i: (0, i)),
            pl.BlockSpec((H, C), lambda i: (0, 0)),
            pl.BlockSpec((1, C), lambda i: (0, 0)),
        ],
        out_specs=pl.BlockSpec((G, C), lambda i: (0, 0)),
        out_shape=jax.ShapeDtypeStruct((G, C), jnp.float32),
        scratch_shapes=[pltpu.VMEM((G, H), jnp.float32)],
    )(o2, hp2T, dinvT, b2c, batch2d, Wfc, bfcc)


def kernel(x, edge_index, batch, W1, b1, W2, b2, Wfc, bfc):
    ei_flat = edge_index.reshape(2 * E)
    degp = _sc_deg()(ei_flat)
    h1preT = _tc_prea(x, W1)
    hp1T, dinvT = _tc_preb(degp, h1preT)
    o1 = _sc_edge()(ei_flat, hp1T)
    hp2T = _tc_mid(o1, hp1T, dinvT, b1.reshape(H, 1), W2.T)
    o2 = _sc_edge()(ei_flat, hp2T)
    return _tc_post(o2, hp2T, dinvT, b2.reshape(H, 1),
                    batch.reshape(1, N), Wfc, bfc.reshape(1, C))


# speedup vs baseline: 1.0173x; 1.0173x over previous
import functools

import jax
import jax.numpy as jnp
from jax import lax
from jax.experimental import pallas as pl
from jax.experimental.pallas import tpu as pltpu
from jax.experimental.pallas import tpu_sc as plsc

N = 10000
E = 320000
F_IN = 128
H = 16
C = 10
G = 128

NB = 1024
GRID_N = 10

NC = 2
NS = 16
NW = NC * NS
FG = 4
EG = NW // FG
FPW = H // FG
EPG = 39936
ECH = 4992
ETAIL = E - EG * EPG
DPW = 9984
DCH = 4992
DTAIL = E - NW * DPW

_mesh = lambda: plsc.VectorSubcoreMesh(
    core_axis_name="c", subcore_axis_name="s", num_cores=NC, num_subcores=NS)


def _sc_deg_body(ei_hbm, out_hbm, ei_buf, acc):
    wid = lax.axis_index("s") * NC + lax.axis_index("c")
    zero = jnp.zeros((16,), jnp.float32)
    ones = jnp.ones((16,), jnp.float32)

    @plsc.parallel_loop(0, N // 16, unroll=8)
    def _(i):
        acc[pl.ds(i * 16, 16)] = zero

    base = wid * DPW
    for ch in range(DPW // DCH):
        pltpu.sync_copy(ei_hbm.at[:, pl.ds(base + ch * DCH, DCH)], ei_buf)

        @plsc.parallel_loop(0, DCH // 16, unroll=8)
        def _(i):
            d16 = ei_buf[1, pl.ds(i * 16, 16)]
            plsc.addupdate_scatter(acc, [d16], ones)

    @pl.when(wid == 0)
    def _():
        pltpu.sync_copy(ei_hbm.at[:, pl.ds(NW * DPW, DTAIL)], ei_buf.at[:, :DTAIL])

        @plsc.parallel_loop(0, DTAIL // 16, unroll=8)
        def _(i):
            d16 = ei_buf[1, pl.ds(i * 16, 16)]
            plsc.addupdate_scatter(acc, [d16], ones)

    pltpu.sync_copy(acc, out_hbm.at[wid])


@functools.cache
def _sc_deg():
    return pl.kernel(
        _sc_deg_body,
        out_type=jax.ShapeDtypeStruct((NW, N), jnp.float32),
        mesh=_mesh(),
        scratch_types=[
            pltpu.VMEM((2, DCH), jnp.int32),
            pltpu.VMEM((N,), jnp.float32),
        ],
        compiler_params=pltpu.CompilerParams(needs_layout_passes=False),
    )


def _sc_edge_body(ei_hbm, hpT_hbm, out_hbm, ei_buf0, ei_buf1, table, acc, sems):
    wid = lax.axis_index("s") * NC + lax.axis_index("c")
    fg = wid // EG
    eg = wid % EG

    pltpu.sync_copy(hpT_hbm.at[pl.ds(fg * FPW, FPW), :], table)

    zero = jnp.zeros((16,), jnp.float32)

    @plsc.parallel_loop(0, N // 16, unroll=8)
    def _(i):
        for j in range(FPW):
            acc[j, pl.ds(i * 16, 16)] = zero

    jv = [jnp.full((16,), j, jnp.int32) for j in range(FPW)]
    base = eg * EPG
    nch = EPG // ECH
    bufs = [ei_buf0, ei_buf1]
    cps = [pltpu.make_async_copy(ei_hbm.at[:, pl.ds(base + ch * ECH, ECH)],
                                 bufs[ch % 2], sems.at[ch % 2])
           for ch in range(nch)]
    cps[0].start()
    for ch in range(nch):
        cps[ch].wait()
        if ch + 1 < nch:
            cps[ch + 1].start()
        eb = bufs[ch % 2]

        @plsc.parallel_loop(0, ECH // 16, unroll=8)
        def _(i):
            s16 = eb[0, pl.ds(i * 16, 16)]
            d16 = eb[1, pl.ds(i * 16, 16)]
            for j in range(FPW):
                v = plsc.load_gather(table, [jv[j], s16])
                plsc.addupdate_scatter(acc, [jv[j], d16], v)

    @pl.when(eg == EG - 1)
    def _():
        pltpu.sync_copy(ei_hbm.at[:, pl.ds(EG * EPG, ETAIL)],
                        ei_buf0.at[:, :ETAIL])

        @plsc.parallel_loop(0, ETAIL // 16, unroll=8)
        def _(i):
            s16 = ei_buf0[0, pl.ds(i * 16, 16)]
            d16 = ei_buf0[1, pl.ds(i * 16, 16)]
            for j in range(FPW):
                v = plsc.load_gather(table, [jv[j], s16])
                plsc.addupdate_scatter(acc, [jv[j], d16], v)

    pltpu.sync_copy(acc, out_hbm.at[eg, pl.ds(fg * FPW, FPW), :])


@functools.cache
def _sc_edge():
    return pl.kernel(
        _sc_edge_body,
        out_type=jax.ShapeDtypeStruct((EG, H, N), jnp.float32),
        mesh=_mesh(),
        scratch_types=[
            pltpu.VMEM((2, ECH), jnp.int32),
            pltpu.VMEM((2, ECH), jnp.int32),
            pltpu.VMEM((FPW, N), jnp.float32),
            pltpu.VMEM((FPW, N), jnp.float32),
            pltpu.SemaphoreType.DMA((2,)),
        ],
        compiler_params=pltpu.CompilerParams(needs_layout_passes=False),
    )


def _tc_prea_body(x_ref, w1_ref, hT_ref):
    h = jnp.dot(x_ref[...], w1_ref[...], preferred_element_type=jnp.float32)
    hT_ref[...] = h.T


def _tc_prea(x, W1):
    return pl.pallas_call(
        _tc_prea_body,
        grid=(GRID_N,),
        in_specs=[
            pl.BlockSpec((NB, F_IN), lambda i: (i, 0)),
            pl.BlockSpec((F_IN, H), lambda i: (0, 0)),
        ],
        out_specs=pl.BlockSpec((H, NB), lambda i: (0, i)),
        out_shape=jax.ShapeDtypeStruct((H, N), jnp.float32),
    )(x, W1)


def _tc_preb_body(degp_ref, hT_ref, hpT_ref, dinvT_ref):
    deg = jnp.sum(degp_ref[...], axis=0) + 1.0
    dinv = lax.rsqrt(deg)[None, :]
    hpT_ref[...] = hT_ref[...] * dinv
    dinvT_ref[...] = dinv


def _tc_preb(degp, h1T):
    return pl.pallas_call(
        _tc_preb_body,
        grid=(GRID_N,),
        in_specs=[
            pl.BlockSpec((NW, NB), lambda i: (0, i)),
            pl.BlockSpec((H, NB), lambda i: (0, i)),
        ],
        out_specs=[
            pl.BlockSpec((H, NB), lambda i: (0, i)),
            pl.BlockSpec((1, NB), lambda i: (0, i)),
        ],
        out_shape=[
            jax.ShapeDtypeStruct((H, N), jnp.float32),
            jax.ShapeDtypeStruct((1, N), jnp.float32),
        ],
    )(degp, h1T)


def _tc_mid_body(o1_ref, hpT_ref, dinvT_ref, b1_ref, w2_ref, hp2T_ref):
    s1 = jnp.sum(o1_ref[...], axis=0)
    dinv = dinvT_ref[...]
    h1 = jnp.maximum(dinv * (s1 + hpT_ref[...]) + b1_ref[...], 0.0)
    h2p = jnp.dot(w2_ref[...], h1, preferred_element_type=jnp.float32)
    hp2T_ref[...] = h2p * dinv


def _tc_mid(o1, hp1T, dinvT, b1c, W2):
    return pl.pallas_call(
        _tc_mid_body,
        grid=(GRID_N,),
        in_specs=[
            pl.BlockSpec((EG, H, NB), lambda i: (0, 0, i)),
            pl.BlockSpec((H, NB), lambda i: (0, i)),
            pl.BlockSpec((1, NB), lambda i: (0, i)),
            pl.BlockSpec((H, 1), lambda i: (0, 0)),
            pl.BlockSpec((H, H), lambda i: (0, 0)),
        ],
        out_specs=pl.BlockSpec((H, NB), lambda i: (0, i)),
        out_shape=jax.ShapeDtypeStruct((H, N), jnp.float32),
    )(o1, hp1T, dinvT, b1c, W2)


def _tc_post_body(o2_ref, hpT_ref, dinvT_ref, b2_ref, batch_ref, wfc_ref,
                  bfc_ref, out_ref, pooled):
    i = pl.program_id(0)
    s2 = jnp.sum(o2_ref[...], axis=0)
    dinv = dinvT_ref[...]
    h2 = dinv * (s2 + hpT_ref[...]) + b2_ref[...]

    b = batch_ref[...]
    gids = lax.broadcasted_iota(jnp.int32, (G, NB), 0)
    nidx = lax.broadcasted_iota(jnp.int32, (G, NB), 1) + i * NB
    mask = (b == gids) & (nidx < N)
    neg = jnp.float32(-jnp.inf)
    cols = [jnp.max(jnp.where(mask, h2[f:f + 1, :], neg), axis=1, keepdims=True)
            for f in range(H)]
    cont = jnp.concatenate(cols, axis=1)

    @pl.when(i == 0)
    def _():
        pooled[...] = jnp.full((G, H), neg, jnp.float32)

    pooled[...] = jnp.maximum(pooled[...], cont)

    @pl.when(i == pl.num_programs(0) - 1)
    def _():
        logits = jnp.dot(pooled[...], wfc_ref[...],
                         preferred_element_type=jnp.float32) + bfc_ref[...]
        m = jnp.max(logits, axis=1, keepdims=True)
        lse = jnp.log(jnp.sum(jnp.exp(logits - m), axis=1, keepdims=True))
        out_ref[...] = logits - m - lse


def _tc_post(o2, hp2T, dinvT, b2c, batch2d, Wfc, bfcc):
    return pl.pallas_call(
        _tc_post_body,
        grid=(GRID_N,),
        in_specs=[
            pl.BlockSpec((EG, H, NB), lambda i: (0, 0, i)),
            pl.BlockSpec((H, NB), lambda i: (0, i)),
            pl.BlockSpec((1, NB), lambda i: (0, i)),
            pl.BlockSpec((H, 1), lambda i: (0, 0)),
            pl.BlockSpec((1, NB), lambda i: (0, i)),
            pl.BlockSpec((H, C), lambda i: (0, 0)),
            pl.BlockSpec((1, C), lambda i: (0, 0)),
        ],
        out_specs=pl.BlockSpec((G, C), lambda i: (0, 0)),
        out_shape=jax.ShapeDtypeStruct((G, C), jnp.float32),
        scratch_shapes=[pltpu.VMEM((G, H), jnp.float32)],
    )(o2, hp2T, dinvT, b2c, batch2d, Wfc, bfcc)


def kernel(x, edge_index, batch, W1, b1, W2, b2, Wfc, bfc):
    degp = _sc_deg()(edge_index)
    h1preT = _tc_prea(x, W1)
    hp1T, dinvT = _tc_preb(degp, h1preT)
    o1 = _sc_edge()(edge_index, hp1T)
    hp2T = _tc_mid(o1, hp1T, dinvT, b1.reshape(H, 1), W2.T)
    o2 = _sc_edge()(edge_index, hp2T)
    return _tc_post(o2, hp2T, dinvT, b2.reshape(H, 1),
                    batch.reshape(1, N), Wfc, bfc.reshape(1, C))
